# baseline (speedup 1.0000x reference)
import jax
import jax.numpy as jnp
from jax.experimental import pallas as pl
from jax.experimental.pallas import tpu as pltpu

_N = 10000
_F = 128
_TM = 200


def _fused_kernel(x_ref, w_ref, adj_ref, out_ref, s_ref):
    @pl.when(pl.program_id(0) == 0)
    def _():
        s_ref[...] = jnp.dot(
            x_ref[...], w_ref[...], preferred_element_type=jnp.float32
        ).astype(jnp.bfloat16)

    a = adj_ref[...].astype(jnp.bfloat16)
    out_ref[...] = jnp.dot(a, s_ref[...], preferred_element_type=jnp.float32)


def kernel(inputs, adj, weight):
    return pl.pallas_call(
        _fused_kernel,
        grid=(_N // _TM,),
        in_specs=[
            pl.BlockSpec((_N, _F), lambda i: (0, 0)),
            pl.BlockSpec((_F, _F), lambda i: (0, 0)),
            pl.BlockSpec((_TM, _N), lambda i: (i, 0)),
        ],
        out_specs=pl.BlockSpec((_TM, _F), lambda i: (i, 0)),
        out_shape=jax.ShapeDtypeStruct((_N, _F), jnp.float32),
        scratch_shapes=[pltpu.VMEM((_N, _F), jnp.bfloat16)],
    )(inputs, weight, adj)


# speedup vs baseline: 1.0108x; 1.0108x over previous
import jax
import jax.numpy as jnp
from jax.experimental import pallas as pl
from jax.experimental.pallas import tpu as pltpu

_N = 10000
_F = 128
_TM = 400


def _fused_kernel(x_ref, w_ref, adj_ref, out_ref, s_ref):
    @pl.when(pl.program_id(0) == 0)
    def _():
        s_ref[...] = jnp.dot(
            x_ref[...], w_ref[...], preferred_element_type=jnp.float32
        )

    out_ref[...] = jnp.dot(
        adj_ref[...], s_ref[...], preferred_element_type=jnp.float32
    )


def kernel(inputs, adj, weight):
    return pl.pallas_call(
        _fused_kernel,
        grid=(_N // _TM,),
        in_specs=[
            pl.BlockSpec((_N, _F), lambda i: (0, 0)),
            pl.BlockSpec((_F, _F), lambda i: (0, 0)),
            pl.BlockSpec((_TM, _N), lambda i: (i, 0)),
        ],
        out_specs=pl.BlockSpec((_TM, _F), lambda i: (i, 0)),
        out_shape=jax.ShapeDtypeStruct((_N, _F), jnp.float32),
        scratch_shapes=[pltpu.VMEM((_N, _F), jnp.float32)],
    )(inputs, weight, adj)
